# Initial kernel scaffold; baseline (speedup 1.0000x reference)
import jax
import jax.numpy as jnp
from jax.experimental import pallas as pl


def kernel(x, neutral_idx, W):
    raise NotImplementedError("write your pallas kernel here")


# speedup vs baseline: 1.0125x; 1.0125x over previous
import functools

import jax
import jax.numpy as jnp
from jax import lax
from jax.experimental import pallas as pl
from jax.experimental.pallas import tpu as pltpu
from jax.experimental.pallas import tpu_sc as plsc

_LN2 = 0.6931471805599453


def kernel(x, neutral_idx, W):
    B = x.shape[0]
    V = W.shape[0]
    Wf = W.reshape(V)

    info = plsc.get_sparse_core_info()
    NC, NS, L = info.num_cores, info.num_subcores, info.num_lanes
    NW = NC * NS
    b_per_w = B // NW
    CH = 128
    K = b_per_w // CH

    x_r = x.reshape(NW, K, CH)
    n_idx = jnp.full((L,), neutral_idx, dtype=jnp.int32)
    mesh = plsc.VectorSubcoreMesh(core_axis_name="c", subcore_axis_name="s")

    @functools.partial(
        pl.kernel,
        mesh=mesh,
        out_type=jax.ShapeDtypeStruct((NW, K, CH), jnp.float32),
        scratch_types=[
            pltpu.VMEM((K, CH), jnp.int32),
            pltpu.VMEM((K, CH), jnp.float32),
            pltpu.VMEM((L,), jnp.int32),
            pltpu.VMEM((L,), jnp.float32),
            pltpu.VMEM((K, CH), jnp.float32),
            pltpu.SemaphoreType.DMA,
        ],
    )
    def run(table_hbm, nidx_hbm, xr_hbm, out_hbm,
            idx_v, vals_v, nidx_v, nval_v, out_v, sem):
        wid = lax.axis_index("s") * NC + lax.axis_index("c")
        pltpu.sync_copy(xr_hbm.at[wid], idx_v)
        pltpu.sync_copy(nidx_hbm, nidx_v)
        copies = [
            pltpu.async_copy(table_hbm.at[idx_v.at[j]], vals_v.at[j], sem)
            for j in range(K)
        ]
        copies.append(pltpu.async_copy(table_hbm.at[nidx_v], nval_v, sem))
        for c in copies:
            c.wait()
        nvec = nval_v[...]
        for j in range(K):
            for i in range(CH // L):
                v = vals_v[j, pl.ds(i * L, L)]
                out_v[j, pl.ds(i * L, L)] = jnp.exp((v - nvec) * _LN2)
        pltpu.sync_copy(out_v, out_hbm.at[wid])

    out = run(Wf, n_idx, x_r)
    return out.reshape(B, 1)
